# Initial kernel scaffold; baseline (speedup 1.0000x reference)
import jax
import jax.numpy as jnp
from jax.experimental import pallas as pl


def kernel(importance_scores):
    raise NotImplementedError("write your pallas kernel here")


# speedup vs baseline: 22.6739x; 22.6739x over previous
import functools

import jax
import jax.numpy as jnp
from jax.experimental import pallas as pl
from jax.experimental.pallas import tpu as pltpu

H = 384
W = 384
N = H * W
TOPK_RATIO = 0.3
K = int(TOPK_RATIO * N)
RANK = N - K
TEMPERATURE = 10.0

ROWS_PER_BLOCK = 8


def _topk_mask_kernel(x_ref, o_ref, keys_ref):
    x = x_ref[...]
    u = jax.lax.bitcast_convert_type(x, jnp.uint32)
    keys_ref[...] = jnp.where(
        u >= jnp.uint32(0x80000000), ~u, u | jnp.uint32(0x80000000)
    )

    def body(t, p):
        b = (jnp.uint32(31) - t.astype(jnp.uint32))
        bit = jnp.uint32(1) << b
        test = p | (bit - jnp.uint32(1))
        c = jnp.sum((keys_ref[...] <= test).astype(jnp.int32), axis=1,
                    keepdims=True)
        return jnp.where(c >= RANK, p, p | bit)

    p0 = jnp.zeros((x.shape[0], 1), jnp.uint32)
    p = jax.lax.fori_loop(0, 32, body, p0)
    u_thr = jnp.where(p >= jnp.uint32(0x80000000), p ^ jnp.uint32(0x80000000),
                      ~p)
    thr = jax.lax.bitcast_convert_type(u_thr, jnp.float32)
    z = TEMPERATURE * (x - thr)
    o_ref[...] = 1.0 / (1.0 + jnp.exp(-z))


@jax.jit
def kernel(importance_scores):
    B, C, _, _ = importance_scores.shape
    flat = importance_scores.reshape(B * C, N)
    R = ROWS_PER_BLOCK
    grid = (B * C) // R
    out = pl.pallas_call(
        _topk_mask_kernel,
        grid=(grid,),
        in_specs=[pl.BlockSpec((R, N), lambda i: (i, 0))],
        out_specs=pl.BlockSpec((R, N), lambda i: (i, 0)),
        out_shape=jax.ShapeDtypeStruct((B * C, N), jnp.float32),
        scratch_shapes=[pltpu.VMEM((R, N), jnp.uint32)],
    )(flat)
    return out.reshape(B, C, H, W)


# speedup vs baseline: 60.4975x; 2.6682x over previous
import jax
import jax.numpy as jnp
from jax.experimental import pallas as pl
from jax.experimental.pallas import tpu as pltpu

H = 384
W = 384
N = H * W
TOPK_RATIO = 0.3
K = int(TOPK_RATIO * N)
RANK = N - K
TEMPERATURE = 10.0

ROWS_PER_BLOCK = 16


def _topk_mask_kernel(x_ref, o_ref, keys_ref):
    u = jax.lax.bitcast_convert_type(x_ref[...], jnp.uint32)
    keys_ref[...] = jnp.where(
        u >= jnp.uint32(0x80000000), ~u, u | jnp.uint32(0x80000000)
    )

    def body(t, p):
        b = (jnp.uint32(31) - t.astype(jnp.uint32))
        bit = jnp.uint32(1) << b
        test = p | (bit - jnp.uint32(1))
        c = jnp.sum((keys_ref[...] <= test).astype(jnp.int32),
                    axis=(1, 2, 3), keepdims=True)
        return jnp.where(c >= RANK, p, p | bit)

    p0 = jnp.zeros((x_ref.shape[0], 1, 1, 1), jnp.uint32)
    p = jax.lax.fori_loop(0, 32, body, p0)
    u_thr = jnp.where(p >= jnp.uint32(0x80000000), p ^ jnp.uint32(0x80000000),
                      ~p)
    thr = jax.lax.bitcast_convert_type(u_thr, jnp.float32)
    z = TEMPERATURE * (x_ref[...] - thr)
    o_ref[...] = 1.0 / (1.0 + jnp.exp(-z))


@jax.jit
def kernel(importance_scores):
    B, C, h, w = importance_scores.shape
    R = ROWS_PER_BLOCK
    grid = (B * C) // R
    x = importance_scores.reshape(B * C, 1, h, w)
    out = pl.pallas_call(
        _topk_mask_kernel,
        grid=(grid,),
        in_specs=[pl.BlockSpec((R, 1, h, w), lambda i: (i, 0, 0, 0))],
        out_specs=pl.BlockSpec((R, 1, h, w), lambda i: (i, 0, 0, 0)),
        out_shape=jax.ShapeDtypeStruct((B * C, 1, h, w), jnp.float32),
        scratch_shapes=[pltpu.VMEM((R, 1, h, w), jnp.uint32)],
    )(x)
    return out.reshape(B, C, h, w)


# speedup vs baseline: 62.0760x; 1.0261x over previous
import jax
import jax.numpy as jnp
from jax.experimental import pallas as pl
from jax.experimental.pallas import tpu as pltpu

H = 384
W = 384
N = H * W
TOPK_RATIO = 0.3
K = int(TOPK_RATIO * N)
RANK = N - K
TEMPERATURE = 10.0

ROWS_PER_BLOCK = 16

def _decode(p):
    sign = jnp.uint32(0x80000000)
    return jax.lax.bitcast_convert_type(
        jnp.where(p >= sign, p ^ sign, ~p), jnp.float32)


def _topk_mask_kernel(x_ref, o_ref):
    def body(t, p):
        b = (jnp.uint32(31) - t.astype(jnp.uint32))
        bit = jnp.uint32(1) << b
        test = _decode(p | (bit - jnp.uint32(1)))
        c = jnp.sum((x_ref[...] <= test).astype(jnp.int32),
                    axis=(1, 2, 3), keepdims=True)
        return jnp.where(c >= RANK, p, p | bit)

    p0 = jnp.zeros((x_ref.shape[0], 1, 1, 1), jnp.uint32)
    p = jax.lax.fori_loop(0, 32, body, p0)
    thr = _decode(p)
    z = TEMPERATURE * (x_ref[...] - thr)
    o_ref[...] = 1.0 / (1.0 + jnp.exp(-z))


@jax.jit
def kernel(importance_scores):
    B, C, h, w = importance_scores.shape
    R = ROWS_PER_BLOCK
    grid = (B * C) // R
    x = importance_scores.reshape(B * C, 1, h, w)
    out = pl.pallas_call(
        _topk_mask_kernel,
        grid=(grid,),
        in_specs=[pl.BlockSpec((R, 1, h, w), lambda i: (i, 0, 0, 0))],
        out_specs=pl.BlockSpec((R, 1, h, w), lambda i: (i, 0, 0, 0)),
        out_shape=jax.ShapeDtypeStruct((B * C, 1, h, w), jnp.float32),
    )(x)
    return out.reshape(B, C, h, w)


# speedup vs baseline: 62.0774x; 1.0000x over previous
import jax
import jax.numpy as jnp
from jax.experimental import pallas as pl
from jax.experimental.pallas import tpu as pltpu

H = 384
W = 384
N = H * W
TOPK_RATIO = 0.3
K = int(TOPK_RATIO * N)
RANK = N - K
TEMPERATURE = 10.0

ROWS_PER_BLOCK = 32

def _decode(p):
    sign = jnp.uint32(0x80000000)
    return jax.lax.bitcast_convert_type(
        jnp.where(p >= sign, p ^ sign, ~p), jnp.float32)


def _topk_mask_kernel(x_ref, o_ref):
    def body(t, p):
        b = (jnp.uint32(31) - t.astype(jnp.uint32))
        bit = jnp.uint32(1) << b
        test = _decode(p | (bit - jnp.uint32(1)))
        c = jnp.sum((x_ref[...] <= test).astype(jnp.int32),
                    axis=(1, 2, 3), keepdims=True)
        return jnp.where(c >= RANK, p, p | bit)

    p0 = jnp.zeros((x_ref.shape[0], 1, 1, 1), jnp.uint32)
    p = jax.lax.fori_loop(0, 32, body, p0)
    thr = _decode(p)
    z = TEMPERATURE * (x_ref[...] - thr)
    o_ref[...] = 1.0 / (1.0 + jnp.exp(-z))


@jax.jit
def kernel(importance_scores):
    B, C, h, w = importance_scores.shape
    R = ROWS_PER_BLOCK
    grid = (B * C) // R
    x = importance_scores.reshape(B * C, 1, h, w)
    out = pl.pallas_call(
        _topk_mask_kernel,
        grid=(grid,),
        in_specs=[pl.BlockSpec((R, 1, h, w), lambda i: (i, 0, 0, 0))],
        out_specs=pl.BlockSpec((R, 1, h, w), lambda i: (i, 0, 0, 0)),
        out_shape=jax.ShapeDtypeStruct((B * C, 1, h, w), jnp.float32),
    )(x)
    return out.reshape(B, C, h, w)
